# baseline (speedup 1.0000x reference)
import functools

import jax
import jax.numpy as jnp
from jax import lax
from jax.experimental import pallas as pl
from jax.experimental.pallas import tpu as pltpu
from jax.experimental.pallas import tpu_sc as plsc

N = 10000
E = 320000
D = 128
G = 64

N_P = 10240
ROWS_PER_TILE = N_P // 16
NC, NS = 2, 16
NW = NC * NS
CHUNK = 128
CPW0 = 120
CPW1 = 40
CHUNKS = NS * (CPW0 + CPW1)
CHUNKS_PAD = CHUNKS + CPW0 - CPW1
E_PAD = CHUNKS_PAD * CHUNK
NBUF = 2

def _make_seg_sum():
  @functools.partial(
      pl.kernel,
      out_type=jax.ShapeDtypeStruct((NC, N_P, D), jnp.float32),
      mesh=plsc.VectorSubcoreMesh(core_axis_name="c", subcore_axis_name="s"),
      scratch_types=[
          pltpu.VMEM((CPW0, CHUNK), jnp.int32),
          pltpu.VMEM((NBUF, 1, CHUNK), jnp.int32),
          pltpu.VMEM((NBUF, CHUNK, D), jnp.float32),
          pltpu.VMEM_SHARED((N_P, D), jnp.float32),
          pltpu.SemaphoreType.DMA((NBUF,)),
          pltpu.SemaphoreType.DMA((NBUF,)),
      ],
  )
  def seg_sum(table_hbm, srcp_hbm, dstp_hbm, zeros_hbm, out_hbm,
              src_v, dst_v, rows_v, acc_sh, gsem, dsem):
    cid = lax.axis_index("c")
    sid = lax.axis_index("s")
    my_cpw = jnp.where(cid == 0, CPW0, CPW1)
    chunk_off = jnp.where(cid == 0, sid * CPW0, NS * CPW0 + sid * CPW1)

    pltpu.sync_copy(zeros_hbm, acc_sh.at[pl.ds(sid * ROWS_PER_TILE,
                                               ROWS_PER_TILE)])
    pltpu.sync_copy(srcp_hbm.at[pl.ds(chunk_off, CPW0)], src_v)
    plsc.subcore_barrier()

    for b in range(NBUF):
      pltpu.async_copy(table_hbm.at[src_v.at[b]], rows_v.at[b], gsem.at[b])
      pltpu.async_copy(dstp_hbm.at[chunk_off + b], dst_v.at[b], dsem.at[b])

    @pl.loop(0, my_cpw)
    def _(j):
      b = lax.rem(j, NBUF)
      pltpu.make_async_copy(table_hbm.at[src_v.at[j]], rows_v.at[b],
                            gsem.at[b]).wait()
      pltpu.make_async_copy(dstp_hbm.at[chunk_off + j], dst_v.at[b],
                            dsem.at[b]).wait()
      pltpu.sync_copy(rows_v.at[b], acc_sh.at[dst_v.at[b, 0]], add=True)
      nj = j + NBUF

      @pl.when(nj < my_cpw)
      def _():
        pltpu.async_copy(table_hbm.at[src_v.at[nj]], rows_v.at[b], gsem.at[b])
        pltpu.async_copy(dstp_hbm.at[chunk_off + nj], dst_v.at[b], dsem.at[b])

    plsc.subcore_barrier()
    rows = pl.ds(sid * ROWS_PER_TILE, ROWS_PER_TILE)
    pltpu.sync_copy(acc_sh.at[rows], out_hbm.at[cid, rows])

  return seg_sum


def _dot(a, b):
  return lax.dot_general(a, b, (((1,), (0,)), ((), ())),
                         preferred_element_type=jnp.float32)


_NB = 8
_BR = N_P // _NB


def _conv_block(agg_ref, x_ref, wrelT_ref, wrootT_ref, b_ref, o_ref):
  agg = agg_ref[0] + agg_ref[1]
  h = _dot(agg, wrelT_ref[...]) + _dot(x_ref[...], wrootT_ref[...]) + b_ref[...]
  o_ref[...] = jnp.maximum(h, 0.0)


def _layer1(agg, x_p, W_rel1, b_rel1, W_root1):
  return pl.pallas_call(
      _conv_block,
      grid=(_NB,),
      in_specs=[
          pl.BlockSpec((2, _BR, D), lambda i: (0, i, 0)),
          pl.BlockSpec((_BR, D), lambda i: (i, 0)),
          pl.BlockSpec((D, D), lambda i: (0, 0)),
          pl.BlockSpec((D, D), lambda i: (0, 0)),
          pl.BlockSpec((1, D), lambda i: (0, 0)),
      ],
      out_specs=pl.BlockSpec((_BR, D), lambda i: (i, 0)),
      out_shape=jax.ShapeDtypeStruct((N_P, D), jnp.float32),
  )(agg, x_p, W_rel1.T, W_root1.T, b_rel1.reshape(1, D))


def _head_block(agg_ref, h1_ref, wrelT_ref, wrootT_ref, b_ref,
                fc1T_ref, fc1b_ref, fc2T_ref, fc2b_ref, batch_ref,
                mid_ref, pool_ref, out_ref):
  i = pl.program_id(0)

  agg = agg_ref[0] + agg_ref[1]
  h2 = _dot(agg, wrelT_ref[...]) + _dot(h1_ref[...], wrootT_ref[...]) + b_ref[...]
  h2 = jnp.maximum(h2, 0.0)

  lane = lax.broadcasted_iota(jnp.int32, (_BR, D), 1)
  m8 = lane < 8
  neg = jnp.float32(-jnp.inf)
  mx = jnp.max(jnp.where(m8, h2, neg), axis=1, keepdims=True)
  e = jnp.where(m8, jnp.exp(h2 - mx), 0.0)
  s = jnp.sum(e, axis=1, keepdims=True)
  mid_ref[...] = e / s

  z = jnp.maximum(_dot(h2, fc1T_ref[...]) + fc1b_ref[...], 0.0)
  z2 = jnp.maximum(_dot(z, fc2T_ref[...]) + fc2b_ref[...], 0.0)

  b = batch_ref[...]
  seg = lax.broadcasted_iota(jnp.int32, (_BR, D), 1).astype(jnp.float32)
  mask = b == seg
  r0 = jnp.max(jnp.where(mask, z2[:, 0:1], neg), axis=0, keepdims=True)
  r1 = jnp.max(jnp.where(mask, z2[:, 1:2], neg), axis=0, keepdims=True)

  row = lax.broadcasted_iota(jnp.int32, (8, D), 0)
  upd = jnp.where(row == 0, r0, jnp.where(row == 1, r1, neg))

  @pl.when(i == 0)
  def _():
    pool_ref[...] = jnp.full((8, D), neg, jnp.float32)

  pool_ref[...] = jnp.maximum(pool_ref[...], upd)

  p = pool_ref[...]
  p0 = p[0:1, :]
  p1 = p[1:2, :]
  pmx = jnp.maximum(p0, p1)
  e0 = jnp.exp(p0 - pmx)
  e1 = jnp.exp(p1 - pmx)
  ps = e0 + e1
  out_ref[...] = jnp.where(row == 0, e0 / ps, jnp.where(row == 1, e1 / ps, 0.0))


def _head(agg, h1, W_rel2, b_rel2, W_root2, fc1T_p, fc1b_p, fc2T_p, fc2b_p,
          batch_2d):
  return pl.pallas_call(
      _head_block,
      grid=(_NB,),
      in_specs=[
          pl.BlockSpec((2, _BR, D), lambda i: (0, i, 0)),
          pl.BlockSpec((_BR, D), lambda i: (i, 0)),
          pl.BlockSpec((D, D), lambda i: (0, 0)),
          pl.BlockSpec((D, D), lambda i: (0, 0)),
          pl.BlockSpec((1, D), lambda i: (0, 0)),
          pl.BlockSpec((D, D), lambda i: (0, 0)),
          pl.BlockSpec((1, D), lambda i: (0, 0)),
          pl.BlockSpec((D, D), lambda i: (0, 0)),
          pl.BlockSpec((1, D), lambda i: (0, 0)),
          pl.BlockSpec((_BR, 1), lambda i: (i, 0)),
      ],
      out_specs=[
          pl.BlockSpec((_BR, D), lambda i: (i, 0)),
          pl.BlockSpec((8, D), lambda i: (0, 0)),
          pl.BlockSpec((8, D), lambda i: (0, 0)),
      ],
      out_shape=[
          jax.ShapeDtypeStruct((N_P, D), jnp.float32),
          jax.ShapeDtypeStruct((8, D), jnp.float32),
          jax.ShapeDtypeStruct((8, D), jnp.float32),
      ],
  )(agg, h1, W_rel2.T, W_root2.T, b_rel2.reshape(1, D),
    fc1T_p, fc1b_p, fc2T_p, fc2b_p, batch_2d)


def kernel(x, edge_index, batch, W_rel1, b_rel1, W_root1, W_rel2, b_rel2,
           W_root2, fc1_W, fc1_b, fc2_W, fc2_b):
  src = edge_index[0].astype(jnp.int32)
  dst = edge_index[1].astype(jnp.int32)
  pad = E_PAD - E
  src_p = jnp.concatenate([src, jnp.zeros((pad,), jnp.int32)])
  src_p = src_p.reshape(CHUNKS_PAD, CHUNK)
  dst_p = jnp.concatenate([dst, jnp.full((pad,), N, jnp.int32)])
  dst_p = dst_p.reshape(CHUNKS_PAD, 1, CHUNK)
  zeros = jnp.zeros((ROWS_PER_TILE, D), jnp.float32)

  x_p = jnp.pad(x, ((0, N_P - N), (0, 0)))

  seg_sum = _make_seg_sum()
  agg1 = seg_sum(x_p, src_p, dst_p, zeros)
  h1 = _layer1(agg1, x_p, W_rel1, b_rel1, W_root1)
  agg2 = seg_sum(h1, src_p, dst_p, zeros)

  fc1T_p = jnp.zeros((D, D), jnp.float32).at[:, :8].set(fc1_W.T)
  fc1b_p = jnp.zeros((1, D), jnp.float32).at[0, :8].set(fc1_b)
  fc2T_p = jnp.zeros((D, D), jnp.float32).at[:8, :2].set(fc2_W.T)
  fc2b_p = jnp.zeros((1, D), jnp.float32).at[0, :2].set(fc2_b)
  batch_2d = jnp.concatenate([batch.astype(jnp.int32),
                              jnp.full((N_P - N,), G, jnp.int32)])
  batch_2d = batch_2d.astype(jnp.float32).reshape(N_P, 1)

  mid_p, _, out_p = _head(agg2, h1, W_rel2, b_rel2, W_root2,
                          fc1T_p, fc1b_p, fc2T_p, fc2b_p, batch_2d)

  mid = mid_p[:N, :8]
  out = out_p[:2, :G].T
  return (mid, out)


# speedup vs baseline: 2.8726x; 2.8726x over previous
import functools

import jax
import jax.numpy as jnp
from jax import lax
from jax.experimental import pallas as pl
from jax.experimental.pallas import tpu as pltpu
from jax.experimental.pallas import tpu_sc as plsc

N = 10000
E = 320000
D = 128
G = 64

N_P = 10240
ROWS_PER_TILE = N_P // 16
NC, NS = 2, 16
NW = NC * NS
CHUNK = 128
CPW0 = 80
CPW1 = 80
CHUNKS = NS * (CPW0 + CPW1)
CHUNKS_PAD = CHUNKS + CPW0 - CPW1
E_PAD = CHUNKS_PAD * CHUNK
NBUF = 2

def _make_seg_sum():
  @functools.partial(
      pl.kernel,
      out_type=jax.ShapeDtypeStruct((NC, N_P, D), jnp.float32),
      mesh=plsc.VectorSubcoreMesh(core_axis_name="c", subcore_axis_name="s"),
      scratch_types=[
          pltpu.VMEM((CPW0, CHUNK), jnp.int32),
          pltpu.VMEM((NBUF, 1, CHUNK), jnp.int32),
          pltpu.VMEM((NBUF, CHUNK, D), jnp.float32),
          pltpu.VMEM_SHARED((N_P, D), jnp.float32),
          pltpu.SemaphoreType.DMA((NBUF,)),
          pltpu.SemaphoreType.DMA((NBUF,)),
      ],
  )
  def seg_sum(table_hbm, srcp_hbm, dstp_hbm, zeros_hbm, out_hbm,
              src_v, dst_v, rows_v, acc_sh, gsem, dsem):
    cid = lax.axis_index("c")
    sid = lax.axis_index("s")
    my_cpw = jnp.where(cid == 0, CPW0, CPW1)
    chunk_off = jnp.where(cid == 0, sid * CPW0, NS * CPW0 + sid * CPW1)

    pltpu.sync_copy(zeros_hbm, acc_sh.at[pl.ds(sid * ROWS_PER_TILE,
                                               ROWS_PER_TILE)])
    pltpu.sync_copy(srcp_hbm.at[pl.ds(chunk_off, CPW0)], src_v)
    plsc.subcore_barrier()

    for b in range(NBUF):
      pltpu.async_copy(table_hbm.at[src_v.at[b]], rows_v.at[b], gsem.at[b])
      pltpu.async_copy(dstp_hbm.at[chunk_off + b], dst_v.at[b], dsem.at[b])

    @pl.loop(0, my_cpw)
    def _(j):
      b = lax.rem(j, NBUF)
      pltpu.make_async_copy(table_hbm.at[src_v.at[j]], rows_v.at[b],
                            gsem.at[b]).wait()
      pltpu.make_async_copy(dstp_hbm.at[chunk_off + j], dst_v.at[b],
                            dsem.at[b]).wait()
      pltpu.sync_copy(rows_v.at[b], acc_sh.at[dst_v.at[b, 0]], add=True)
      nj = j + NBUF

      @pl.when(nj < my_cpw)
      def _():
        pltpu.async_copy(table_hbm.at[src_v.at[nj]], rows_v.at[b], gsem.at[b])
        pltpu.async_copy(dstp_hbm.at[chunk_off + nj], dst_v.at[b], dsem.at[b])

    plsc.subcore_barrier()
    rows = pl.ds(sid * ROWS_PER_TILE, ROWS_PER_TILE)
    pltpu.sync_copy(acc_sh.at[rows], out_hbm.at[cid, rows])

  return seg_sum


def _dot(a, b):
  return lax.dot_general(a, b, (((1,), (0,)), ((), ())),
                         preferred_element_type=jnp.float32)


_NB = 8
_BR = N_P // _NB


def _conv_block(agg_ref, x_ref, wrelT_ref, wrootT_ref, b_ref, o_ref):
  agg = agg_ref[0] + agg_ref[1]
  h = _dot(agg, wrelT_ref[...]) + _dot(x_ref[...], wrootT_ref[...]) + b_ref[...]
  o_ref[...] = jnp.maximum(h, 0.0)


def _layer1(agg, x_p, W_rel1, b_rel1, W_root1):
  return pl.pallas_call(
      _conv_block,
      grid=(_NB,),
      in_specs=[
          pl.BlockSpec((2, _BR, D), lambda i: (0, i, 0)),
          pl.BlockSpec((_BR, D), lambda i: (i, 0)),
          pl.BlockSpec((D, D), lambda i: (0, 0)),
          pl.BlockSpec((D, D), lambda i: (0, 0)),
          pl.BlockSpec((1, D), lambda i: (0, 0)),
      ],
      out_specs=pl.BlockSpec((_BR, D), lambda i: (i, 0)),
      out_shape=jax.ShapeDtypeStruct((N_P, D), jnp.float32),
  )(agg, x_p, W_rel1.T, W_root1.T, b_rel1.reshape(1, D))


def _head_block(agg_ref, h1_ref, wrelT_ref, wrootT_ref, b_ref,
                fc1T_ref, fc1b_ref, fc2T_ref, fc2b_ref, batch_ref,
                mid_ref, pool_ref, out_ref):
  i = pl.program_id(0)

  agg = agg_ref[0] + agg_ref[1]
  h2 = _dot(agg, wrelT_ref[...]) + _dot(h1_ref[...], wrootT_ref[...]) + b_ref[...]
  h2 = jnp.maximum(h2, 0.0)

  lane = lax.broadcasted_iota(jnp.int32, (_BR, D), 1)
  m8 = lane < 8
  neg = jnp.float32(-jnp.inf)
  mx = jnp.max(jnp.where(m8, h2, neg), axis=1, keepdims=True)
  e = jnp.where(m8, jnp.exp(h2 - mx), 0.0)
  s = jnp.sum(e, axis=1, keepdims=True)
  mid_ref[...] = e / s

  z = jnp.maximum(_dot(h2, fc1T_ref[...]) + fc1b_ref[...], 0.0)
  z2 = jnp.maximum(_dot(z, fc2T_ref[...]) + fc2b_ref[...], 0.0)

  b = batch_ref[...]
  seg = lax.broadcasted_iota(jnp.int32, (_BR, D), 1).astype(jnp.float32)
  mask = b == seg
  r0 = jnp.max(jnp.where(mask, z2[:, 0:1], neg), axis=0, keepdims=True)
  r1 = jnp.max(jnp.where(mask, z2[:, 1:2], neg), axis=0, keepdims=True)

  row = lax.broadcasted_iota(jnp.int32, (8, D), 0)
  upd = jnp.where(row == 0, r0, jnp.where(row == 1, r1, neg))

  @pl.when(i == 0)
  def _():
    pool_ref[...] = jnp.full((8, D), neg, jnp.float32)

  pool_ref[...] = jnp.maximum(pool_ref[...], upd)

  p = pool_ref[...]
  p0 = p[0:1, :]
  p1 = p[1:2, :]
  pmx = jnp.maximum(p0, p1)
  e0 = jnp.exp(p0 - pmx)
  e1 = jnp.exp(p1 - pmx)
  ps = e0 + e1
  out_ref[...] = jnp.where(row == 0, e0 / ps, jnp.where(row == 1, e1 / ps, 0.0))


def _head(agg, h1, W_rel2, b_rel2, W_root2, fc1T_p, fc1b_p, fc2T_p, fc2b_p,
          batch_2d):
  return pl.pallas_call(
      _head_block,
      grid=(_NB,),
      in_specs=[
          pl.BlockSpec((2, _BR, D), lambda i: (0, i, 0)),
          pl.BlockSpec((_BR, D), lambda i: (i, 0)),
          pl.BlockSpec((D, D), lambda i: (0, 0)),
          pl.BlockSpec((D, D), lambda i: (0, 0)),
          pl.BlockSpec((1, D), lambda i: (0, 0)),
          pl.BlockSpec((D, D), lambda i: (0, 0)),
          pl.BlockSpec((1, D), lambda i: (0, 0)),
          pl.BlockSpec((D, D), lambda i: (0, 0)),
          pl.BlockSpec((1, D), lambda i: (0, 0)),
          pl.BlockSpec((_BR, 1), lambda i: (i, 0)),
      ],
      out_specs=[
          pl.BlockSpec((_BR, D), lambda i: (i, 0)),
          pl.BlockSpec((8, D), lambda i: (0, 0)),
          pl.BlockSpec((8, D), lambda i: (0, 0)),
      ],
      out_shape=[
          jax.ShapeDtypeStruct((N_P, D), jnp.float32),
          jax.ShapeDtypeStruct((8, D), jnp.float32),
          jax.ShapeDtypeStruct((8, D), jnp.float32),
      ],
  )(agg, h1, W_rel2.T, W_root2.T, b_rel2.reshape(1, D),
    fc1T_p, fc1b_p, fc2T_p, fc2b_p, batch_2d)


def kernel(x, edge_index, batch, W_rel1, b_rel1, W_root1, W_rel2, b_rel2,
           W_root2, fc1_W, fc1_b, fc2_W, fc2_b):
  src = edge_index[0].astype(jnp.int32)
  dst = edge_index[1].astype(jnp.int32)
  pad = E_PAD - E
  padv = jnp.arange(pad, dtype=jnp.int32)
  src_p = jnp.concatenate([src, padv % N])
  src_p = src_p.reshape(CHUNKS_PAD, CHUNK)
  dst_p = jnp.concatenate([dst, N + padv % (N_P - N)])
  dst_p = dst_p.reshape(CHUNKS_PAD, 1, CHUNK)
  zeros = jnp.zeros((ROWS_PER_TILE, D), jnp.float32)

  x_p = jnp.pad(x, ((0, N_P - N), (0, 0)))

  seg_sum = _make_seg_sum()
  agg1 = seg_sum(x_p, src_p, dst_p, zeros)
  h1 = _layer1(agg1, x_p, W_rel1, b_rel1, W_root1)
  agg2 = seg_sum(h1, src_p, dst_p, zeros)

  fc1T_p = jnp.zeros((D, D), jnp.float32).at[:, :8].set(fc1_W.T)
  fc1b_p = jnp.zeros((1, D), jnp.float32).at[0, :8].set(fc1_b)
  fc2T_p = jnp.zeros((D, D), jnp.float32).at[:8, :2].set(fc2_W.T)
  fc2b_p = jnp.zeros((1, D), jnp.float32).at[0, :2].set(fc2_b)
  batch_2d = jnp.concatenate([batch.astype(jnp.int32),
                              jnp.full((N_P - N,), G, jnp.int32)])
  batch_2d = batch_2d.astype(jnp.float32).reshape(N_P, 1)

  mid_p, _, out_p = _head(agg2, h1, W_rel2, b_rel2, W_root2,
                          fc1T_p, fc1b_p, fc2T_p, fc2b_p, batch_2d)

  mid = mid_p[:N, :8]
  out = out_p[:2, :G].T
  return (mid, out)


# speedup vs baseline: 2.9429x; 1.0245x over previous
import functools

import jax
import jax.numpy as jnp
from jax import lax
from jax.experimental import pallas as pl
from jax.experimental.pallas import tpu as pltpu
from jax.experimental.pallas import tpu_sc as plsc

N = 10000
E = 320000
D = 128
G = 64

N_P = 10240
ROWS_PER_TILE = N_P // 16
NC, NS = 2, 16
NW = NC * NS
CHUNK = 96
CPW = 112
CHUNKS = NW * CPW
CHUNKS_PAD = CHUNKS
E_PAD = CHUNKS_PAD * CHUNK
NBUF = 3

def _make_seg_sum():
  @functools.partial(
      pl.kernel,
      out_type=jax.ShapeDtypeStruct((NC, N_P, D), jnp.float32),
      mesh=plsc.VectorSubcoreMesh(core_axis_name="c", subcore_axis_name="s"),
      scratch_types=[
          pltpu.VMEM((CPW * CHUNK,), jnp.int32),
          pltpu.VMEM((NBUF, 1, CHUNK), jnp.int32),
          pltpu.VMEM((NBUF, CHUNK, D), jnp.float32),
          pltpu.VMEM_SHARED((N_P, D), jnp.float32),
          pltpu.SemaphoreType.DMA((NBUF,)),
          pltpu.SemaphoreType.DMA((NBUF,)),
          pltpu.SemaphoreType.DMA((NBUF,)),
      ],
  )
  def seg_sum(table_hbm, srcp_hbm, dstp_hbm, zeros_hbm, out_hbm,
              src_v, dst_v, rows_v, acc_sh, gsem, dsem, ssem):
    cid = lax.axis_index("c")
    sid = lax.axis_index("s")
    wid = sid * NC + cid
    chunk_off = wid * CPW
    src_of = lambda j: src_v.at[pl.ds(j * CHUNK, CHUNK)]

    pltpu.sync_copy(zeros_hbm, acc_sh.at[pl.ds(sid * ROWS_PER_TILE,
                                               ROWS_PER_TILE)])
    pltpu.sync_copy(srcp_hbm.at[pl.ds(chunk_off * CHUNK, CPW * CHUNK)], src_v)
    plsc.subcore_barrier()

    for b in range(2):
      pltpu.async_copy(table_hbm.at[src_of(b)], rows_v.at[b], gsem.at[b])
      pltpu.async_copy(dstp_hbm.at[chunk_off + b], dst_v.at[b], dsem.at[b])

    @pl.loop(0, CPW)
    def _(j):
      b = lax.rem(j, NBUF)
      p = lax.rem(j + 2, NBUF)
      pltpu.make_async_copy(table_hbm.at[src_of(j)], rows_v.at[b],
                            gsem.at[b]).wait()
      pltpu.make_async_copy(dstp_hbm.at[chunk_off + j], dst_v.at[b],
                            dsem.at[b]).wait()
      pltpu.async_copy(rows_v.at[b], acc_sh.at[dst_v.at[b, 0]], ssem.at[b],
                       add=True)

      @pl.when(j >= 1)
      def _():
        pltpu.make_async_copy(rows_v.at[p], acc_sh.at[dst_v.at[p, 0]],
                              ssem.at[p]).wait()
      nj = j + 2

      @pl.when(nj < CPW)
      def _():
        pltpu.async_copy(table_hbm.at[src_of(nj)], rows_v.at[p], gsem.at[p])
        pltpu.async_copy(dstp_hbm.at[chunk_off + nj], dst_v.at[p], dsem.at[p])

    lb = (CPW - 1) % NBUF
    pltpu.make_async_copy(rows_v.at[lb], acc_sh.at[dst_v.at[lb, 0]],
                          ssem.at[lb]).wait()

    plsc.subcore_barrier()
    rows = pl.ds(sid * ROWS_PER_TILE, ROWS_PER_TILE)
    pltpu.sync_copy(acc_sh.at[rows], out_hbm.at[cid, rows])

  return seg_sum


def _dot(a, b):
  return lax.dot_general(a, b, (((1,), (0,)), ((), ())),
                         preferred_element_type=jnp.float32)


_NB = 8
_BR = N_P // _NB


def _conv_block(agg_ref, x_ref, wrelT_ref, wrootT_ref, b_ref, o_ref):
  agg = agg_ref[0] + agg_ref[1]
  h = _dot(agg, wrelT_ref[...]) + _dot(x_ref[...], wrootT_ref[...]) + b_ref[...]
  o_ref[...] = jnp.maximum(h, 0.0)


def _layer1(agg, x_p, W_rel1, b_rel1, W_root1):
  return pl.pallas_call(
      _conv_block,
      grid=(_NB,),
      in_specs=[
          pl.BlockSpec((2, _BR, D), lambda i: (0, i, 0)),
          pl.BlockSpec((_BR, D), lambda i: (i, 0)),
          pl.BlockSpec((D, D), lambda i: (0, 0)),
          pl.BlockSpec((D, D), lambda i: (0, 0)),
          pl.BlockSpec((1, D), lambda i: (0, 0)),
      ],
      out_specs=pl.BlockSpec((_BR, D), lambda i: (i, 0)),
      out_shape=jax.ShapeDtypeStruct((N_P, D), jnp.float32),
  )(agg, x_p, W_rel1.T, W_root1.T, b_rel1.reshape(1, D))


def _head_block(agg_ref, h1_ref, wrelT_ref, wrootT_ref, b_ref,
                fc1T_ref, fc1b_ref, fc2T_ref, fc2b_ref, batch_ref,
                mid_ref, pool_ref, out_ref):
  i = pl.program_id(0)

  agg = agg_ref[0] + agg_ref[1]
  h2 = _dot(agg, wrelT_ref[...]) + _dot(h1_ref[...], wrootT_ref[...]) + b_ref[...]
  h2 = jnp.maximum(h2, 0.0)

  lane = lax.broadcasted_iota(jnp.int32, (_BR, D), 1)
  m8 = lane < 8
  neg = jnp.float32(-jnp.inf)
  mx = jnp.max(jnp.where(m8, h2, neg), axis=1, keepdims=True)
  e = jnp.where(m8, jnp.exp(h2 - mx), 0.0)
  s = jnp.sum(e, axis=1, keepdims=True)
  mid_ref[...] = e / s

  z = jnp.maximum(_dot(h2, fc1T_ref[...]) + fc1b_ref[...], 0.0)
  z2 = jnp.maximum(_dot(z, fc2T_ref[...]) + fc2b_ref[...], 0.0)

  b = batch_ref[...]
  seg = lax.broadcasted_iota(jnp.int32, (_BR, D), 1).astype(jnp.float32)
  mask = b == seg
  r0 = jnp.max(jnp.where(mask, z2[:, 0:1], neg), axis=0, keepdims=True)
  r1 = jnp.max(jnp.where(mask, z2[:, 1:2], neg), axis=0, keepdims=True)

  row = lax.broadcasted_iota(jnp.int32, (8, D), 0)
  upd = jnp.where(row == 0, r0, jnp.where(row == 1, r1, neg))

  @pl.when(i == 0)
  def _():
    pool_ref[...] = jnp.full((8, D), neg, jnp.float32)

  pool_ref[...] = jnp.maximum(pool_ref[...], upd)

  p = pool_ref[...]
  p0 = p[0:1, :]
  p1 = p[1:2, :]
  pmx = jnp.maximum(p0, p1)
  e0 = jnp.exp(p0 - pmx)
  e1 = jnp.exp(p1 - pmx)
  ps = e0 + e1
  out_ref[...] = jnp.where(row == 0, e0 / ps, jnp.where(row == 1, e1 / ps, 0.0))


def _head(agg, h1, W_rel2, b_rel2, W_root2, fc1T_p, fc1b_p, fc2T_p, fc2b_p,
          batch_2d):
  return pl.pallas_call(
      _head_block,
      grid=(_NB,),
      in_specs=[
          pl.BlockSpec((2, _BR, D), lambda i: (0, i, 0)),
          pl.BlockSpec((_BR, D), lambda i: (i, 0)),
          pl.BlockSpec((D, D), lambda i: (0, 0)),
          pl.BlockSpec((D, D), lambda i: (0, 0)),
          pl.BlockSpec((1, D), lambda i: (0, 0)),
          pl.BlockSpec((D, D), lambda i: (0, 0)),
          pl.BlockSpec((1, D), lambda i: (0, 0)),
          pl.BlockSpec((D, D), lambda i: (0, 0)),
          pl.BlockSpec((1, D), lambda i: (0, 0)),
          pl.BlockSpec((_BR, 1), lambda i: (i, 0)),
      ],
      out_specs=[
          pl.BlockSpec((_BR, D), lambda i: (i, 0)),
          pl.BlockSpec((8, D), lambda i: (0, 0)),
          pl.BlockSpec((8, D), lambda i: (0, 0)),
      ],
      out_shape=[
          jax.ShapeDtypeStruct((N_P, D), jnp.float32),
          jax.ShapeDtypeStruct((8, D), jnp.float32),
          jax.ShapeDtypeStruct((8, D), jnp.float32),
      ],
  )(agg, h1, W_rel2.T, W_root2.T, b_rel2.reshape(1, D),
    fc1T_p, fc1b_p, fc2T_p, fc2b_p, batch_2d)


def kernel(x, edge_index, batch, W_rel1, b_rel1, W_root1, W_rel2, b_rel2,
           W_root2, fc1_W, fc1_b, fc2_W, fc2_b):
  src = edge_index[0].astype(jnp.int32)
  dst = edge_index[1].astype(jnp.int32)
  pad = E_PAD - E
  padv = jnp.arange(pad, dtype=jnp.int32)
  src_p = jnp.concatenate([src, padv % N])
  dst_p = jnp.concatenate([dst, N + padv % (N_P - N)])
  dst_p = dst_p.reshape(CHUNKS_PAD, 1, CHUNK)
  zeros = jnp.zeros((ROWS_PER_TILE, D), jnp.float32)

  x_p = jnp.pad(x, ((0, N_P - N), (0, 0)))

  seg_sum = _make_seg_sum()
  agg1 = seg_sum(x_p, src_p, dst_p, zeros)
  h1 = _layer1(agg1, x_p, W_rel1, b_rel1, W_root1)
  agg2 = seg_sum(h1, src_p, dst_p, zeros)

  fc1T_p = jnp.zeros((D, D), jnp.float32).at[:, :8].set(fc1_W.T)
  fc1b_p = jnp.zeros((1, D), jnp.float32).at[0, :8].set(fc1_b)
  fc2T_p = jnp.zeros((D, D), jnp.float32).at[:8, :2].set(fc2_W.T)
  fc2b_p = jnp.zeros((1, D), jnp.float32).at[0, :2].set(fc2_b)
  batch_2d = jnp.concatenate([batch.astype(jnp.int32),
                              jnp.full((N_P - N,), G, jnp.int32)])
  batch_2d = batch_2d.astype(jnp.float32).reshape(N_P, 1)

  mid_p, _, out_p = _head(agg2, h1, W_rel2, b_rel2, W_root2,
                          fc1T_p, fc1b_p, fc2T_p, fc2b_p, batch_2d)

  mid = mid_p[:N, :8]
  out = out_p[:2, :G].T
  return (mid, out)
